# Initial kernel scaffold; baseline (speedup 1.0000x reference)
import jax
import jax.numpy as jnp
from jax.experimental import pallas as pl


def kernel(indices, values, shape, b):
    raise NotImplementedError("write your pallas kernel here")


# speedup vs baseline: 6.5563x; 6.5563x over previous
import functools

import jax
import jax.numpy as jnp
from jax import lax
from jax.experimental import pallas as pl
from jax.experimental.pallas import tpu as pltpu
from jax.experimental.pallas import tpu_sc as plsc

LANES = 16
NC = 2
NS = 16
NW = NC * NS
COLS = 64
CPT = COLS // NW
CH = 8192


def _spmm_sc(rows, cols, vals, bT, n, nchunk):
    mesh = plsc.VectorSubcoreMesh(
        core_axis_name="c", subcore_axis_name="s",
        num_cores=NC, num_subcores=NS)

    @functools.partial(
        pl.kernel,
        out_type=jax.ShapeDtypeStruct((COLS, n), jnp.float32),
        mesh=mesh,
        scratch_types=[
            pltpu.VMEM((CH,), jnp.int32),
            pltpu.VMEM((CH,), jnp.int32),
            pltpu.VMEM((CH,), jnp.int32),
            pltpu.VMEM((CH,), jnp.int32),
            pltpu.VMEM((CH,), jnp.float32),
            pltpu.VMEM((CH,), jnp.float32),
            pltpu.VMEM((n,), jnp.float32),
            pltpu.VMEM((n,), jnp.float32),
            pltpu.VMEM((n,), jnp.float32),
            pltpu.VMEM((n,), jnp.float32),
            pltpu.SemaphoreType.DMA,
            pltpu.SemaphoreType.DMA,
            pltpu.SemaphoreType.DMA,
        ],
        compiler_params=pltpu.CompilerParams(needs_layout_passes=False),
    )
    def k(rows_hbm, cols_hbm, vals_hbm, bT_hbm, out_hbm,
          r0, r1, c0, c1, v0, v1, bA, bB, aA, aB, semb, sem0, sem1):
        one = jnp.int32(1)
        wid = lax.axis_index("s") * jnp.int32(NC) + lax.axis_index("c")
        col0 = wid * jnp.int32(CPT)

        pltpu.async_copy(bT_hbm.at[col0], bA, semb)
        pltpu.async_copy(bT_hbm.at[col0 + one], bB, semb)

        rbufs, cbufs, vbufs, sems = (r0, r1), (c0, c1), (v0, v1), (sem0, sem1)

        def fire(g, slot):
            base = g * jnp.int32(CH)
            pltpu.async_copy(rows_hbm.at[pl.ds(base, CH)], rbufs[slot], sems[slot])
            pltpu.async_copy(cols_hbm.at[pl.ds(base, CH)], cbufs[slot], sems[slot])
            pltpu.async_copy(vals_hbm.at[pl.ds(base, CH)], vbufs[slot], sems[slot])

        def drain(slot):
            pltpu.make_async_copy(rows_hbm.at[pl.ds(jnp.int32(0), CH)], rbufs[slot], sems[slot]).wait()
            pltpu.make_async_copy(cols_hbm.at[pl.ds(jnp.int32(0), CH)], cbufs[slot], sems[slot]).wait()
            pltpu.make_async_copy(vals_hbm.at[pl.ds(jnp.int32(0), CH)], vbufs[slot], sems[slot]).wait()

        fire(jnp.int32(0), 0)

        zeros = jnp.zeros((LANES,), jnp.float32)

        def zbody(i, carry):
            off = i * jnp.int32(LANES)
            aA[pl.ds(off, LANES)] = zeros
            aB[pl.ds(off, LANES)] = zeros
            return carry

        lax.fori_loop(jnp.int32(0), jnp.int32(n // LANES), zbody, 0)

        pltpu.make_async_copy(bT_hbm.at[jnp.int32(0)], bA, semb).wait()
        pltpu.make_async_copy(bT_hbm.at[jnp.int32(0)], bB, semb).wait()

        def process(slot):
            rbuf, cbuf, vbuf = rbufs[slot], cbufs[slot], vbufs[slot]

            def step(i, carry):
                off = i * jnp.int32(LANES)
                r = rbuf[pl.ds(off, LANES)]
                c = cbuf[pl.ds(off, LANES)]
                v = vbuf[pl.ds(off, LANES)]
                plsc.addupdate_scatter(aA, [r], plsc.load_gather(bA, [c]) * v)
                plsc.addupdate_scatter(aB, [r], plsc.load_gather(bB, [c]) * v)
                return carry

            lax.fori_loop(jnp.int32(0), jnp.int32(CH // LANES), step, 0)

        def outer(g2, carry):
            for b in range(2):
                g = g2 * jnp.int32(2) + jnp.int32(b)

                @pl.when(g + one < jnp.int32(nchunk))
                def _():
                    fire(g + one, 1 - b)

                drain(b)
                process(b)
            return carry

        lax.fori_loop(jnp.int32(0), jnp.int32(nchunk // 2), outer, 0)

        pltpu.sync_copy(aA, out_hbm.at[col0])
        pltpu.sync_copy(aB, out_hbm.at[col0 + one])

    return k(rows, cols, vals, bT)


def kernel(indices, values, shape, b):
    n = b.shape[0]
    idx = indices.astype(jnp.int32)
    rows, cols = idx[0], idx[1]
    vals = values.astype(jnp.float32)
    nnz = vals.shape[0]
    pad = (-nnz) % (2 * CH)
    if pad:
        rows = jnp.pad(rows, (0, pad))
        cols = jnp.pad(cols, (0, pad))
        vals = jnp.pad(vals, (0, pad))
    nchunk = (nnz + pad) // CH
    bT = jnp.asarray(b, jnp.float32).T
    outT = _spmm_sc(rows, cols, vals, bT, n, nchunk)
    return outT.T.astype(b.dtype)


# speedup vs baseline: 6.7081x; 1.0232x over previous
import functools

import jax
import jax.numpy as jnp
from jax import lax
from jax.experimental import pallas as pl
from jax.experimental.pallas import tpu as pltpu
from jax.experimental.pallas import tpu_sc as plsc

LANES = 16
NC = 2
NS = 16
NW = NC * NS
COLS = 64
CPT = COLS // NW
CH = 8192


def _spmm_sc(rows, cols, vals, bT, n, nchunk):
    mesh = plsc.VectorSubcoreMesh(
        core_axis_name="c", subcore_axis_name="s",
        num_cores=NC, num_subcores=NS)

    @functools.partial(
        pl.kernel,
        out_type=jax.ShapeDtypeStruct((COLS, n), jnp.float32),
        mesh=mesh,
        scratch_types=[
            pltpu.VMEM((CH,), jnp.int32),
            pltpu.VMEM((CH,), jnp.int32),
            pltpu.VMEM((CH,), jnp.int32),
            pltpu.VMEM((CH,), jnp.int32),
            pltpu.VMEM((CH,), jnp.float32),
            pltpu.VMEM((CH,), jnp.float32),
            pltpu.VMEM((n,), jnp.float32),
            pltpu.VMEM((n,), jnp.float32),
            pltpu.VMEM((n,), jnp.float32),
            pltpu.VMEM((n,), jnp.float32),
            pltpu.SemaphoreType.DMA,
            pltpu.SemaphoreType.DMA,
            pltpu.SemaphoreType.DMA,
        ],
        compiler_params=pltpu.CompilerParams(needs_layout_passes=False),
    )
    def k(rows_hbm, cols_hbm, vals_hbm, bT_hbm, out_hbm,
          r0, r1, c0, c1, v0, v1, bA, bB, aA, aB, semb, sem0, sem1):
        one = jnp.int32(1)
        wid = lax.axis_index("s") * jnp.int32(NC) + lax.axis_index("c")
        col0 = wid * jnp.int32(CPT)

        pltpu.async_copy(bT_hbm.at[col0], bA, semb)
        pltpu.async_copy(bT_hbm.at[col0 + one], bB, semb)

        rbufs, cbufs, vbufs, sems = (r0, r1), (c0, c1), (v0, v1), (sem0, sem1)

        def fire(g, slot):
            base = g * jnp.int32(CH)
            pltpu.async_copy(rows_hbm.at[pl.ds(base, CH)], rbufs[slot], sems[slot])
            pltpu.async_copy(cols_hbm.at[pl.ds(base, CH)], cbufs[slot], sems[slot])
            pltpu.async_copy(vals_hbm.at[pl.ds(base, CH)], vbufs[slot], sems[slot])

        def drain(slot):
            pltpu.make_async_copy(rows_hbm.at[pl.ds(jnp.int32(0), CH)], rbufs[slot], sems[slot]).wait()
            pltpu.make_async_copy(cols_hbm.at[pl.ds(jnp.int32(0), CH)], cbufs[slot], sems[slot]).wait()
            pltpu.make_async_copy(vals_hbm.at[pl.ds(jnp.int32(0), CH)], vbufs[slot], sems[slot]).wait()

        fire(jnp.int32(0), 0)

        zeros = jnp.zeros((LANES,), jnp.float32)

        def zbody(i, carry):
            off = i * jnp.int32(LANES)
            aA[pl.ds(off, LANES)] = zeros
            aB[pl.ds(off, LANES)] = zeros
            return carry

        lax.fori_loop(jnp.int32(0), jnp.int32(n // LANES), zbody, 0)

        pltpu.make_async_copy(bT_hbm.at[jnp.int32(0)], bA, semb).wait()
        pltpu.make_async_copy(bT_hbm.at[jnp.int32(0)], bB, semb).wait()

        U = 8

        def process(slot):
            rbuf, cbuf, vbuf = rbufs[slot], cbufs[slot], vbufs[slot]

            def step(i, carry):
                base = i * jnp.int32(LANES * U)
                for u in range(U):
                    off = base + jnp.int32(u * LANES)
                    r = rbuf[pl.ds(off, LANES)]
                    c = cbuf[pl.ds(off, LANES)]
                    v = vbuf[pl.ds(off, LANES)]
                    plsc.addupdate_scatter(aA, [r], plsc.load_gather(bA, [c]) * v)
                    plsc.addupdate_scatter(aB, [r], plsc.load_gather(bB, [c]) * v)
                return carry

            lax.fori_loop(jnp.int32(0), jnp.int32(CH // (LANES * U)), step, 0)

        def outer(g2, carry):
            for b in range(2):
                g = g2 * jnp.int32(2) + jnp.int32(b)

                @pl.when(g + one < jnp.int32(nchunk))
                def _():
                    fire(g + one, 1 - b)

                drain(b)
                process(b)
            return carry

        lax.fori_loop(jnp.int32(0), jnp.int32(nchunk // 2), outer, 0)

        pltpu.sync_copy(aA, out_hbm.at[col0])
        pltpu.sync_copy(aB, out_hbm.at[col0 + one])

    return k(rows, cols, vals, bT)


def kernel(indices, values, shape, b):
    n = b.shape[0]
    idx = indices.astype(jnp.int32)
    rows, cols = idx[0], idx[1]
    vals = values.astype(jnp.float32)
    nnz = vals.shape[0]
    pad = (-nnz) % (2 * CH)
    if pad:
        rows = jnp.pad(rows, (0, pad))
        cols = jnp.pad(cols, (0, pad))
        vals = jnp.pad(vals, (0, pad))
    nchunk = (nnz + pad) // CH
    bT = jnp.asarray(b, jnp.float32).T
    outT = _spmm_sc(rows, cols, vals, bT, n, nchunk)
    return outT.T.astype(b.dtype)


# speedup vs baseline: 16.6213x; 2.4778x over previous
import functools

import jax
import jax.numpy as jnp
from jax import lax
from jax.experimental import pallas as pl
from jax.experimental.pallas import tpu as pltpu
from jax.experimental.pallas import tpu_sc as plsc

LANES = 16
NC = 2
NS = 16
NW = NC * NS
COLS = 64
CPT = COLS // NW
CH = 8192


def _spmm_sc(rows, cols, vals, bT, n, nchunk):
    mesh = plsc.VectorSubcoreMesh(
        core_axis_name="c", subcore_axis_name="s",
        num_cores=NC, num_subcores=NS)

    @functools.partial(
        pl.kernel,
        out_type=jax.ShapeDtypeStruct((COLS, n), jnp.float32),
        mesh=mesh,
        scratch_types=[
            pltpu.VMEM((CH,), jnp.int32),
            pltpu.VMEM((CH,), jnp.int32),
            pltpu.VMEM((CH,), jnp.int32),
            pltpu.VMEM((CH,), jnp.int32),
            pltpu.VMEM((CH,), jnp.float32),
            pltpu.VMEM((CH,), jnp.float32),
            pltpu.VMEM((n,), jnp.float32),
            pltpu.VMEM((n,), jnp.float32),
            pltpu.VMEM((n,), jnp.float32),
            pltpu.VMEM((n,), jnp.float32),
            pltpu.SemaphoreType.DMA,
            pltpu.SemaphoreType.DMA,
            pltpu.SemaphoreType.DMA,
        ],
        compiler_params=pltpu.CompilerParams(needs_layout_passes=False),
    )
    def k(rows_hbm, cols_hbm, vals_hbm, bT_hbm, out_hbm,
          r0, r1, c0, c1, v0, v1, bA, bB, aA, aB, semb, sem0, sem1):
        one = jnp.int32(1)
        wid = lax.axis_index("s") * jnp.int32(NC) + lax.axis_index("c")
        col0 = wid * jnp.int32(CPT)

        pltpu.async_copy(bT_hbm.at[col0], bA, semb)
        pltpu.async_copy(bT_hbm.at[col0 + one], bB, semb)

        rbufs, cbufs, vbufs, sems = (r0, r1), (c0, c1), (v0, v1), (sem0, sem1)

        def fire(g, slot):
            base = g * jnp.int32(CH)
            pltpu.async_copy(rows_hbm.at[pl.ds(base, CH)], rbufs[slot], sems[slot])
            pltpu.async_copy(cols_hbm.at[pl.ds(base, CH)], cbufs[slot], sems[slot])
            pltpu.async_copy(vals_hbm.at[pl.ds(base, CH)], vbufs[slot], sems[slot])

        def drain(slot):
            pltpu.make_async_copy(rows_hbm.at[pl.ds(jnp.int32(0), CH)], rbufs[slot], sems[slot]).wait()
            pltpu.make_async_copy(cols_hbm.at[pl.ds(jnp.int32(0), CH)], cbufs[slot], sems[slot]).wait()
            pltpu.make_async_copy(vals_hbm.at[pl.ds(jnp.int32(0), CH)], vbufs[slot], sems[slot]).wait()

        fire(jnp.int32(0), 0)

        zeros = jnp.zeros((LANES,), jnp.float32)

        def zbody(i, carry):
            off = i * jnp.int32(LANES)
            aA[pl.ds(off, LANES)] = zeros
            aB[pl.ds(off, LANES)] = zeros
            return carry

        lax.fori_loop(jnp.int32(0), jnp.int32(n // LANES), zbody, 0)

        pltpu.make_async_copy(bT_hbm.at[jnp.int32(0)], bA, semb).wait()
        pltpu.make_async_copy(bT_hbm.at[jnp.int32(0)], bB, semb).wait()

        def process(slot):
            rbuf, cbuf, vbuf = rbufs[slot], cbufs[slot], vbufs[slot]

            @plsc.parallel_loop(jnp.int32(0), jnp.int32(CH // LANES), jnp.int32(1), unroll=8)
            def _(i):
                off = i * jnp.int32(LANES)
                r = rbuf[pl.ds(off, LANES)]
                c = cbuf[pl.ds(off, LANES)]
                v = vbuf[pl.ds(off, LANES)]
                plsc.addupdate_scatter(aA, [r], plsc.load_gather(bA, [c]) * v)
                plsc.addupdate_scatter(aB, [r], plsc.load_gather(bB, [c]) * v)

        def outer(g2, carry):
            for b in range(2):
                g = g2 * jnp.int32(2) + jnp.int32(b)

                @pl.when(g + one < jnp.int32(nchunk))
                def _():
                    fire(g + one, 1 - b)

                drain(b)
                process(b)
            return carry

        lax.fori_loop(jnp.int32(0), jnp.int32(nchunk // 2), outer, 0)

        pltpu.sync_copy(aA, out_hbm.at[col0])
        pltpu.sync_copy(aB, out_hbm.at[col0 + one])

    return k(rows, cols, vals, bT)


def kernel(indices, values, shape, b):
    n = b.shape[0]
    idx = indices.astype(jnp.int32)
    rows, cols = idx[0], idx[1]
    vals = values.astype(jnp.float32)
    nnz = vals.shape[0]
    pad = (-nnz) % (2 * CH)
    if pad:
        rows = jnp.pad(rows, (0, pad))
        cols = jnp.pad(cols, (0, pad))
        vals = jnp.pad(vals, (0, pad))
    nchunk = (nnz + pad) // CH
    bT = jnp.asarray(b, jnp.float32).T
    outT = _spmm_sc(rows, cols, vals, bT, n, nchunk)
    return outT.T.astype(b.dtype)
